# Initial kernel scaffold; baseline (speedup 1.0000x reference)
import jax
import jax.numpy as jnp
from jax.experimental import pallas as pl


def kernel(x, edge_index, Ws, bs):
    raise NotImplementedError("write your pallas kernel here")


# speedup vs baseline: 7.3503x; 7.3503x over previous
import functools

import jax
import jax.numpy as jnp
from jax import lax
from jax.experimental import pallas as pl
from jax.experimental.pallas import tpu as pltpu
from jax.experimental.pallas import tpu_sc as plsc

_N = 10000
_D = 128
_E = 320000
_L = 20
_NS = 16
_NC = 2
_C = 80
_EPS = _E // _NC
_EPT = _EPS // _NS
_STEPS = _EPT // _C
assert _STEPS * _C == _EPT
_RB = 640
_RL = _N - (_NS - 1) * _RB
_BR = 1000


_sc_mesh = plsc.VectorSubcoreMesh(
    core_axis_name="c", subcore_axis_name="s", num_cores=_NC, num_subcores=_NS
)


@functools.partial(
    pl.kernel,
    mesh=_sc_mesh,
    out_type=[
        jax.ShapeDtypeStruct((_N, _D), jnp.float32),
        jax.ShapeDtypeStruct((_N, _D), jnp.float32),
    ],
    scratch_types=[
        pltpu.VMEM((_C,), jnp.int32),
        pltpu.VMEM((_C,), jnp.int32),
        pltpu.VMEM((_C, _D), jnp.float32),
        pltpu.VMEM_SHARED((_N, _D), jnp.float32),
        pltpu.SemaphoreType.DMA,
    ],
)
def _sc_agg(g_hbm, src_hbm, dst_hbm, out0, out1, sidx, didx, rows, acc, sem):
    cid = lax.axis_index("c")
    sid = lax.axis_index("s")

    @pl.when(sid < _NS - 1)
    def _():
        pltpu.sync_copy(g_hbm.at[pl.ds(sid * _RB, _RB)],
                        acc.at[pl.ds(sid * _RB, _RB)])

    @pl.when(sid == _NS - 1)
    def _():
        pltpu.sync_copy(g_hbm.at[pl.ds((_NS - 1) * _RB, _RL)],
                        acc.at[pl.ds((_NS - 1) * _RB, _RL)])

    plsc.subcore_barrier()

    base = cid * _EPS + sid * _EPT

    def body(i, carry):
        off = base + i * _C
        pltpu.sync_copy(src_hbm.at[pl.ds(off, _C)], sidx)
        pltpu.sync_copy(dst_hbm.at[pl.ds(off, _C)], didx)
        pltpu.async_copy(g_hbm.at[sidx], rows, sem).wait()
        pltpu.sync_copy(rows, acc.at[didx], add=True)
        return carry

    lax.fori_loop(0, _STEPS, body, 0)
    plsc.subcore_barrier()

    def writeback(out_hbm):
        @pl.when(sid < _NS - 1)
        def _():
            pltpu.sync_copy(acc.at[pl.ds(sid * _RB, _RB)],
                            out_hbm.at[pl.ds(sid * _RB, _RB)])

        @pl.when(sid == _NS - 1)
        def _():
            pltpu.sync_copy(acc.at[pl.ds((_NS - 1) * _RB, _RL)],
                            out_hbm.at[pl.ds((_NS - 1) * _RB, _RL)])

    @pl.when(cid == 0)
    def _():
        writeback(out0)

    @pl.when(cid == 1)
    def _():
        writeback(out1)


def _tc_prep_body(d0_ref, d1_ref, o_ref):
    deg = jnp.min(d0_ref[...] + d1_ref[...], axis=1, keepdims=True) - 1.0
    o_ref[...] = lax.rsqrt(deg)


_tc_prep = pl.pallas_call(
    _tc_prep_body,
    grid=(_N // _BR,),
    in_specs=[
        pl.BlockSpec((_BR, _D), lambda i: (i, 0)),
        pl.BlockSpec((_BR, _D), lambda i: (i, 0)),
    ],
    out_specs=pl.BlockSpec((_BR, 1), lambda i: (i, 0)),
    out_shape=jax.ShapeDtypeStruct((_N, 1), jnp.float32),
)


def _dot(a, b):
    return jnp.dot(a.astype(jnp.bfloat16), b.astype(jnp.bfloat16),
                   preferred_element_type=jnp.float32)


def _tc_first_body(x_ref, dinv_ref, w_ref, g_ref):
    g = _dot(x_ref[...], w_ref[...])
    g_ref[...] = g * dinv_ref[...]


_tc_first = pl.pallas_call(
    _tc_first_body,
    grid=(_N // _BR,),
    in_specs=[
        pl.BlockSpec((_BR, _D), lambda i: (i, 0)),
        pl.BlockSpec((_BR, 1), lambda i: (i, 0)),
        pl.BlockSpec((_D, _D), lambda i: (0, 0)),
    ],
    out_specs=pl.BlockSpec((_BR, _D), lambda i: (i, 0)),
    out_shape=jax.ShapeDtypeStruct((_N, _D), jnp.float32),
)


def _tc_mid_body(p0_ref, p1_ref, g_ref, dinv_ref, w_ref, b_ref, go_ref):
    dinv = dinv_ref[...]
    p = p0_ref[...] + p1_ref[...] - g_ref[...]
    h = jnp.maximum(p * dinv + b_ref[...], 0.0)
    g = _dot(h, w_ref[...])
    go_ref[...] = g * dinv


_tc_mid = pl.pallas_call(
    _tc_mid_body,
    grid=(_N // _BR,),
    in_specs=[
        pl.BlockSpec((_BR, _D), lambda i: (i, 0)),
        pl.BlockSpec((_BR, _D), lambda i: (i, 0)),
        pl.BlockSpec((_BR, _D), lambda i: (i, 0)),
        pl.BlockSpec((_BR, 1), lambda i: (i, 0)),
        pl.BlockSpec((_D, _D), lambda i: (0, 0)),
        pl.BlockSpec((1, _D), lambda i: (0, 0)),
    ],
    out_specs=pl.BlockSpec((_BR, _D), lambda i: (i, 0)),
    out_shape=jax.ShapeDtypeStruct((_N, _D), jnp.float32),
)


def _tc_final_body(p0_ref, p1_ref, g_ref, dinv_ref, b_ref, o_ref):
    p = p0_ref[...] + p1_ref[...] - g_ref[...]
    o_ref[...] = p * dinv_ref[...] + b_ref[...]


_tc_final = pl.pallas_call(
    _tc_final_body,
    grid=(_N // _BR,),
    in_specs=[
        pl.BlockSpec((_BR, _D), lambda i: (i, 0)),
        pl.BlockSpec((_BR, _D), lambda i: (i, 0)),
        pl.BlockSpec((_BR, _D), lambda i: (i, 0)),
        pl.BlockSpec((_BR, 1), lambda i: (i, 0)),
        pl.BlockSpec((1, _D), lambda i: (0, 0)),
    ],
    out_specs=pl.BlockSpec((_BR, _D), lambda i: (i, 0)),
    out_shape=jax.ShapeDtypeStruct((_N, _D), jnp.float32),
)


def kernel(x, edge_index, Ws, bs):
    src = edge_index[0]
    dst = edge_index[1]

    ones = jnp.ones((_N, _D), jnp.float32)
    d0, d1 = _sc_agg(ones, src, dst)
    dinv = _tc_prep(d0, d1)

    g = _tc_first(x, dinv, Ws[0])
    for l in range(1, _L):
        p0, p1 = _sc_agg(g, src, dst)
        g = _tc_mid(p0, p1, g, dinv, Ws[l], bs[l - 1].reshape(1, _D))
    p0, p1 = _sc_agg(g, src, dst)
    return _tc_final(p0, p1, g, dinv, bs[_L - 1].reshape(1, _D))


# speedup vs baseline: 16.3004x; 2.2176x over previous
import functools

import jax
import jax.numpy as jnp
from jax import lax
from jax.experimental import pallas as pl
from jax.experimental.pallas import tpu as pltpu
from jax.experimental.pallas import tpu_sc as plsc

_N = 10000
_D = 128
_E = 320000
_L = 20
_NS = 16
_NC = 2
_C = 125
_EPS = _E // _NC
_EPT = _EPS // _NS
_CH = _EPT // _C
_CHH = _CH // 2
assert _CH * _C == _EPT and _CHH % 2 == 0
_RB = 640
_RL = _N - (_NS - 1) * _RB
_BR = 1000


_sc_mesh = plsc.VectorSubcoreMesh(
    core_axis_name="c", subcore_axis_name="s", num_cores=_NC, num_subcores=_NS
)


@functools.partial(
    pl.kernel,
    mesh=_sc_mesh,
    out_type=[
        jax.ShapeDtypeStruct((_N, _D), jnp.float32),
        jax.ShapeDtypeStruct((_N, _D), jnp.float32),
    ],
    scratch_types=[
        pltpu.VMEM((_CHH, _C), jnp.int32),
        pltpu.VMEM((_CHH, _C), jnp.int32),
        pltpu.VMEM((2, _C, _D), jnp.float32),
        pltpu.VMEM_SHARED((_N, _D), jnp.float32),
        pltpu.SemaphoreType.DMA,
    ],
)
def _sc_agg(g_hbm, src_hbm, dst_hbm, out0, out1, sidx, didx, rows, acc, sem):
    cid = lax.axis_index("c")
    sid = lax.axis_index("s")

    @pl.when(sid < _NS - 1)
    def _():
        pltpu.sync_copy(g_hbm.at[pl.ds(sid * _RB, _RB)],
                        acc.at[pl.ds(sid * _RB, _RB)])

    @pl.when(sid == _NS - 1)
    def _():
        pltpu.sync_copy(g_hbm.at[pl.ds((_NS - 1) * _RB, _RL)],
                        acc.at[pl.ds((_NS - 1) * _RB, _RL)])

    plsc.subcore_barrier()

    for h in range(2):
        pltpu.sync_copy(src_hbm.at[cid, sid, pl.ds(h * _CHH, _CHH)], sidx)
        pltpu.sync_copy(dst_hbm.at[cid, sid, pl.ds(h * _CHH, _CHH)], didx)
        pltpu.async_copy(g_hbm.at[sidx.at[0]], rows.at[0], sem)

        def body(k, carry):
            for j in range(2):
                c = 2 * k + j
                pltpu.make_async_copy(g_hbm.at[sidx.at[c]], rows.at[j],
                                      sem).wait()
                if j == 0:
                    pltpu.async_copy(g_hbm.at[sidx.at[c + 1]], rows.at[1], sem)
                else:
                    @pl.when(k < _CHH // 2 - 1)
                    def _():
                        pltpu.async_copy(g_hbm.at[sidx.at[c + 1]], rows.at[0],
                                         sem)
                pltpu.sync_copy(rows.at[j], acc.at[didx.at[c]], add=True)
            return carry

        lax.fori_loop(0, _CHH // 2, body, 0)
    plsc.subcore_barrier()

    def writeback(out_hbm):
        @pl.when(sid < _NS - 1)
        def _():
            pltpu.sync_copy(acc.at[pl.ds(sid * _RB, _RB)],
                            out_hbm.at[pl.ds(sid * _RB, _RB)])

        @pl.when(sid == _NS - 1)
        def _():
            pltpu.sync_copy(acc.at[pl.ds((_NS - 1) * _RB, _RL)],
                            out_hbm.at[pl.ds((_NS - 1) * _RB, _RL)])

    @pl.when(cid == 0)
    def _():
        writeback(out0)

    @pl.when(cid == 1)
    def _():
        writeback(out1)


def _tc_prep_body(d0_ref, d1_ref, o_ref):
    deg = jnp.min(d0_ref[...] + d1_ref[...], axis=1, keepdims=True) - 1.0
    o_ref[...] = lax.rsqrt(deg)


_tc_prep = pl.pallas_call(
    _tc_prep_body,
    grid=(_N // _BR,),
    in_specs=[
        pl.BlockSpec((_BR, _D), lambda i: (i, 0)),
        pl.BlockSpec((_BR, _D), lambda i: (i, 0)),
    ],
    out_specs=pl.BlockSpec((_BR, 1), lambda i: (i, 0)),
    out_shape=jax.ShapeDtypeStruct((_N, 1), jnp.float32),
)


def _dot(a, b):
    return jnp.dot(a.astype(jnp.bfloat16), b.astype(jnp.bfloat16),
                   preferred_element_type=jnp.float32)


def _tc_first_body(x_ref, dinv_ref, w_ref, g_ref):
    g = _dot(x_ref[...], w_ref[...])
    g_ref[...] = g * dinv_ref[...]


_tc_first = pl.pallas_call(
    _tc_first_body,
    grid=(_N // _BR,),
    in_specs=[
        pl.BlockSpec((_BR, _D), lambda i: (i, 0)),
        pl.BlockSpec((_BR, 1), lambda i: (i, 0)),
        pl.BlockSpec((_D, _D), lambda i: (0, 0)),
    ],
    out_specs=pl.BlockSpec((_BR, _D), lambda i: (i, 0)),
    out_shape=jax.ShapeDtypeStruct((_N, _D), jnp.float32),
)


def _tc_mid_body(p0_ref, p1_ref, g_ref, dinv_ref, w_ref, b_ref, go_ref):
    dinv = dinv_ref[...]
    p = p0_ref[...] + p1_ref[...] - g_ref[...]
    h = jnp.maximum(p * dinv + b_ref[...], 0.0)
    g = _dot(h, w_ref[...])
    go_ref[...] = g * dinv


_tc_mid = pl.pallas_call(
    _tc_mid_body,
    grid=(_N // _BR,),
    in_specs=[
        pl.BlockSpec((_BR, _D), lambda i: (i, 0)),
        pl.BlockSpec((_BR, _D), lambda i: (i, 0)),
        pl.BlockSpec((_BR, _D), lambda i: (i, 0)),
        pl.BlockSpec((_BR, 1), lambda i: (i, 0)),
        pl.BlockSpec((_D, _D), lambda i: (0, 0)),
        pl.BlockSpec((1, _D), lambda i: (0, 0)),
    ],
    out_specs=pl.BlockSpec((_BR, _D), lambda i: (i, 0)),
    out_shape=jax.ShapeDtypeStruct((_N, _D), jnp.float32),
)


def _tc_final_body(p0_ref, p1_ref, g_ref, dinv_ref, b_ref, o_ref):
    p = p0_ref[...] + p1_ref[...] - g_ref[...]
    o_ref[...] = p * dinv_ref[...] + b_ref[...]


_tc_final = pl.pallas_call(
    _tc_final_body,
    grid=(_N // _BR,),
    in_specs=[
        pl.BlockSpec((_BR, _D), lambda i: (i, 0)),
        pl.BlockSpec((_BR, _D), lambda i: (i, 0)),
        pl.BlockSpec((_BR, _D), lambda i: (i, 0)),
        pl.BlockSpec((_BR, 1), lambda i: (i, 0)),
        pl.BlockSpec((1, _D), lambda i: (0, 0)),
    ],
    out_specs=pl.BlockSpec((_BR, _D), lambda i: (i, 0)),
    out_shape=jax.ShapeDtypeStruct((_N, _D), jnp.float32),
)


def kernel(x, edge_index, Ws, bs):
    src = edge_index[0].reshape(_NC, _NS, _CH, _C)
    dst = edge_index[1].reshape(_NC, _NS, _CH, _C)

    ones = jnp.ones((_N, _D), jnp.float32)
    d0, d1 = _sc_agg(ones, src, dst)
    dinv = _tc_prep(d0, d1)

    g = _tc_first(x, dinv, Ws[0])
    for l in range(1, _L):
        p0, p1 = _sc_agg(g, src, dst)
        g = _tc_mid(p0, p1, g, dinv, Ws[l], bs[l - 1].reshape(1, _D))
    p0, p1 = _sc_agg(g, src, dst)
    return _tc_final(p0, p1, g, dinv, bs[_L - 1].reshape(1, _D))
